# baseline (speedup 1.0000x reference)
import functools

import numpy as np
import jax
import jax.numpy as jnp
from jax import lax
from jax.experimental import pallas as pl
from jax.experimental.pallas import tpu as pltpu
from jax.experimental.pallas import tpu_sc as plsc

_NUM_LEVELS = 16
_BASE_RES = 16
_MAX_RES = 2048
_HASHMAP_SIZE = 2 ** 19
_GROWTH = np.exp((np.log(_MAX_RES) - np.log(_BASE_RES)) / (_NUM_LEVELS - 1))
_RES = [int(np.floor(_BASE_RES * _GROWTH ** i)) for i in range(_NUM_LEVELS)]
_TS = [min(r * r, _HASHMAP_SIZE) for r in _RES]
_P1 = 2654435761
_P2 = 805459861
_BATCH = 1048576

_NC, _NS, _L = 2, 16, 16
_NW = _NC * _NS
_PW = _BATCH // _NW
_C = 512
_NCHUNK = _PW // _C
_NV = _C // _L
_NSMALL = 7


def _i32c(v):
    return jnp.int32(np.int32(np.uint32(v & 0xFFFFFFFF)))


def _hash_hi(v, ph, plo):
    c16 = jnp.int32(16)
    a = v * ph
    b = v * plo
    return lax.shift_right_logical(a + lax.shift_right_logical(b, c16), c16)


def _combine_pow2(lox, loy, mask):
    return (lox ^ loy) & mask


def _combine_general(lox, hix, loy, hiy, t, c31, c32):
    lo = lox ^ loy
    hi = hix ^ hiy
    nn = lo & jnp.int32(0x7FFFFFFF)
    sb = lax.shift_right_logical(lo, jnp.int32(31))
    acc = lax.rem(nn, jnp.int32(t)) + sb * jnp.int32(c31) + hi * jnp.int32(c32)
    return lax.rem(acc, jnp.int32(t))


@functools.lru_cache(maxsize=1)
def _build():
    mesh = plsc.VectorSubcoreMesh(core_axis_name="c", subcore_axis_name="s")

    p1h, p1l = _P1 >> 16, _P1 & 0xFFFF
    p2h, p2l = _P2 >> 16, _P2 & 0xFFFF

    scratch = []
    for l in range(_NSMALL):
        scratch.append(pltpu.VMEM((_TS[l],), jnp.float32))
        scratch.append(pltpu.VMEM((_TS[l],), jnp.float32))
    scratch += [pltpu.VMEM((_C,), jnp.float32)] * 4
    scratch += [pltpu.VMEM((_C,), jnp.int32)] * 4
    scratch += [pltpu.VMEM((_C,), jnp.int32)] * 4
    scratch += [pltpu.VMEM((_C, 8), jnp.float32)] * 4
    scratch.append(pltpu.VMEM((_C, 2 * _NUM_LEVELS), jnp.float32))
    scratch.append(pltpu.SemaphoreType.DMA)

    @functools.partial(
        pl.kernel,
        out_type=jax.ShapeDtypeStruct((_BATCH, 2 * _NUM_LEVELS), jnp.float32),
        mesh=mesh,
        compiler_params=pltpu.CompilerParams(
            needs_layout_passes=False, use_tc_tiling_on_sc=False),
        scratch_types=scratch,
    )
    def hash_embed(*refs):
        xt_ref = refs[0]
        ta_hbm = refs[1:1 + _NSMALL]
        tb_hbm = refs[1 + _NSMALL:1 + 2 * _NSMALL]
        big_hbm = {_NSMALL + i: r for i, r in enumerate(
            refs[1 + 2 * _NSMALL:1 + 2 * _NSMALL + (_NUM_LEVELS - _NSMALL)])}
        out_ref = refs[1 + 2 * _NSMALL + (_NUM_LEVELS - _NSMALL)]
        sc = list(refs[2 + 2 * _NSMALL + (_NUM_LEVELS - _NSMALL):])
        st_a = [sc[2 * l] for l in range(_NSMALL)]
        st_b = [sc[2 * l + 1] for l in range(_NSMALL)]
        sc = sc[2 * _NSMALL:]
        xn, yn, wx, wy = sc[0:4]
        idx_refs = sc[4:8]
        sub_refs = sc[8:12]
        row_refs = sc[12:16]
        ob = sc[16]
        sem = sc[17]

        wid = (lax.axis_index("s").astype(jnp.int32) * jnp.int32(_NC)
               + lax.axis_index("c").astype(jnp.int32))
        iota = lax.iota(jnp.int32, _L)

        for l in range(_NSMALL):
            pltpu.sync_copy(ta_hbm[l], st_a[l])
            pltpu.sync_copy(tb_hbm[l], st_b[l])

        def level_consts(lvl):
            t = _TS[lvl]
            return dict(
                res_f=jnp.float32(_RES[lvl]),
                res_m1=jnp.int32(_RES[lvl] - 1),
                t=t,
                pow2=(t & (t - 1)) == 0,
                c31=(1 << 31) % t,
                c32=(1 << 32) % t,
                mask=jnp.int32(t - 1),
                p1c=_i32c(_P1),
                p2c=_i32c(_P2),
            )

        def corners(sl, lc):
            xnv = xn[sl]
            ynv = yn[sl]
            xs = xnv * lc["res_f"]
            ys = ynv * lc["res_f"]
            x0 = xs.astype(jnp.int32)
            y0 = ys.astype(jnp.int32)
            wxv = xs - x0.astype(jnp.float32)
            wyv = ys - y0.astype(jnp.float32)
            x1 = jnp.minimum(x0 + jnp.int32(1), lc["res_m1"])
            y1 = jnp.minimum(y0 + jnp.int32(1), lc["res_m1"])
            x0 = jnp.minimum(x0, lc["res_m1"])
            y0 = jnp.minimum(y0, lc["res_m1"])
            lox0 = x0 * lc["p1c"]
            lox1 = x1 * lc["p1c"]
            loy0 = y0 * lc["p2c"]
            loy1 = y1 * lc["p2c"]
            if lc["pow2"]:
                c00 = _combine_pow2(lox0, loy0, lc["mask"])
                c10 = _combine_pow2(lox1, loy0, lc["mask"])
                c01 = _combine_pow2(lox0, loy1, lc["mask"])
                c11 = _combine_pow2(lox1, loy1, lc["mask"])
            else:
                hix0 = _hash_hi(x0, p1h, p1l)
                hix1 = _hash_hi(x1, p1h, p1l)
                hiy0 = _hash_hi(y0, p2h, p2l)
                hiy1 = _hash_hi(y1, p2h, p2l)
                t, c31, c32 = lc["t"], lc["c31"], lc["c32"]
                c00 = _combine_general(lox0, hix0, loy0, hiy0, t, c31, c32)
                c10 = _combine_general(lox1, hix1, loy0, hiy0, t, c31, c32)
                c01 = _combine_general(lox0, hix0, loy1, hiy1, t, c31, c32)
                c11 = _combine_general(lox1, hix1, loy1, hiy1, t, c31, c32)
            return wxv, wyv, c00, c10, c01, c11

        def lerp2(e00, e10, e01, e11, wxv, wyv):
            top = e00 + (e10 - e00) * wxv
            bot = e01 + (e11 - e01) * wxv
            return top + (bot - top) * wyv

        def p0(i, carry):
            s = pl.multiple_of(i * jnp.int32(_L), _L)
            sl = pl.ds(s, _L)
            xn[sl] = jnp.clip((xn[sl] + 1.0) * 0.5, 0.0, 1.0)
            yn[sl] = jnp.clip((yn[sl] + 1.0) * 0.5, 0.0, 1.0)
            return carry

        def make_small(lvl):
            lc = level_consts(lvl)
            ca = jnp.full((_L,), 2 * lvl, jnp.int32)
            cb = jnp.full((_L,), 2 * lvl + 1, jnp.int32)
            ta, tb = st_a[lvl], st_b[lvl]

            def ps(i, carry):
                s = pl.multiple_of(i * jnp.int32(_L), _L)
                sl = pl.ds(s, _L)
                pidx = s + iota
                wxv, wyv, c00, c10, c01, c11 = corners(sl, lc)
                e00a = plsc.load_gather(ta, [c00])
                e10a = plsc.load_gather(ta, [c10])
                e01a = plsc.load_gather(ta, [c01])
                e11a = plsc.load_gather(ta, [c11])
                e00b = plsc.load_gather(tb, [c00])
                e10b = plsc.load_gather(tb, [c10])
                e01b = plsc.load_gather(tb, [c01])
                e11b = plsc.load_gather(tb, [c11])
                plsc.store_scatter(ob, [pidx, ca],
                                   lerp2(e00a, e10a, e01a, e11a, wxv, wyv))
                plsc.store_scatter(ob, [pidx, cb],
                                   lerp2(e00b, e10b, e01b, e11b, wxv, wyv))
                return carry

            return ps

        def make_p1(lvl):
            lc = level_consts(lvl)
            two = jnp.int32(2)
            three = jnp.int32(3)
            one = jnp.int32(1)

            def p1(i, carry):
                s = pl.multiple_of(i * jnp.int32(_L), _L)
                sl = pl.ds(s, _L)
                wxv, wyv, c00, c10, c01, c11 = corners(sl, lc)
                wx[sl] = wxv
                wy[sl] = wyv
                for k, c in enumerate((c00, c10, c01, c11)):
                    idx_refs[k][sl] = lax.shift_right_logical(c, two)
                    sub_refs[k][sl] = lax.shift_left(c & three, one)
                return carry

            return p1

        def make_p2(lvl):
            ca = jnp.full((_L,), 2 * lvl, jnp.int32)
            cb = jnp.full((_L,), 2 * lvl + 1, jnp.int32)
            one = jnp.int32(1)

            def p2(i, carry):
                s = pl.multiple_of(i * jnp.int32(_L), _L)
                sl = pl.ds(s, _L)
                pidx = s + iota
                wxv = wx[sl]
                wyv = wy[sl]
                ea = []
                eb = []
                for k in range(4):
                    sv = sub_refs[k][sl]
                    ea.append(plsc.load_gather(row_refs[k], [iota + s, sv]))
                    eb.append(plsc.load_gather(row_refs[k], [iota + s, sv + one]))
                va = lerp2(ea[0], ea[1], ea[2], ea[3], wxv, wyv)
                vb = lerp2(eb[0], eb[1], eb[2], eb[3], wxv, wyv)
                plsc.store_scatter(ob, [pidx, ca], va)
                plsc.store_scatter(ob, [pidx, cb], vb)
                return carry

            return p2

        smalls = [make_small(l) for l in range(_NSMALL)]
        p1s = {l: make_p1(l) for l in range(_NSMALL, _NUM_LEVELS)}
        p2s = {l: make_p2(l) for l in range(_NSMALL, _NUM_LEVELS)}

        def chunk(g, carry):
            base = pl.multiple_of(wid * jnp.int32(_PW) + g * jnp.int32(_C), _C)
            pltpu.sync_copy(xt_ref.at[jnp.int32(0), pl.ds(base, _C)], xn)
            pltpu.sync_copy(xt_ref.at[jnp.int32(1), pl.ds(base, _C)], yn)
            lax.fori_loop(jnp.int32(0), jnp.int32(_NV), p0, 0)
            for l in range(_NSMALL):
                lax.fori_loop(jnp.int32(0), jnp.int32(_NV), smalls[l], 0)
            for l in range(_NSMALL, _NUM_LEVELS):
                lax.fori_loop(jnp.int32(0), jnp.int32(_NV), p1s[l], 0)
                cps = [pltpu.async_copy(big_hbm[l].at[idx_refs[k]],
                                        row_refs[k], sem)
                       for k in range(4)]
                for cp in cps:
                    cp.wait()
                lax.fori_loop(jnp.int32(0), jnp.int32(_NV), p2s[l], 0)
            pltpu.sync_copy(ob, out_ref.at[pl.ds(base, _C)])
            return carry

        lax.fori_loop(jnp.int32(0), jnp.int32(_NCHUNK), chunk, 0)

    return hash_embed


def kernel(x, tables):
    xt = x.T
    tas = tuple(tables[l][:, 0] for l in range(_NSMALL))
    tbs = tuple(tables[l][:, 1] for l in range(_NSMALL))
    bigs = []
    for l in range(_NSMALL, _NUM_LEVELS):
        flat = tables[l].reshape(-1)
        pad = (-flat.shape[0]) % 8
        if pad:
            flat = jnp.concatenate([flat, jnp.zeros((pad,), flat.dtype)])
        bigs.append(flat.reshape(-1, 8))
    return _build()(xt, *tas, *tbs, *bigs)


# speedup vs baseline: 1.1130x; 1.1130x over previous
import functools

import numpy as np
import jax
import jax.numpy as jnp
from jax import lax
from jax.experimental import pallas as pl
from jax.experimental.pallas import tpu as pltpu
from jax.experimental.pallas import tpu_sc as plsc

_NUM_LEVELS = 16
_BASE_RES = 16
_MAX_RES = 2048
_HASHMAP_SIZE = 2 ** 19
_GROWTH = np.exp((np.log(_MAX_RES) - np.log(_BASE_RES)) / (_NUM_LEVELS - 1))
_RES = [int(np.floor(_BASE_RES * _GROWTH ** i)) for i in range(_NUM_LEVELS)]
_TS = [min(r * r, _HASHMAP_SIZE) for r in _RES]
_P1 = 2654435761
_P2 = 805459861
_BATCH = 1048576

_NC, _NS, _L = 2, 16, 16
_NW = _NC * _NS
_PW = _BATCH // _NW
_C = 512
_NCHUNK = _PW // _C
_NV = _C // _L
_NSMALL = 7


def _i32c(v):
    return jnp.int32(np.int32(np.uint32(v & 0xFFFFFFFF)))


def _hash_hi(v, ph, plo):
    c16 = jnp.int32(16)
    a = v * ph
    b = v * plo
    return lax.shift_right_logical(a + lax.shift_right_logical(b, c16), c16)


def _combine_pow2(lox, loy, mask):
    return (lox ^ loy) & mask


def _combine_general(lox, hix, loy, hiy, t, c31, c32):
    lo = lox ^ loy
    hi = hix ^ hiy
    nn = lo & jnp.int32(0x7FFFFFFF)
    sb = lax.shift_right_logical(lo, jnp.int32(31))
    acc = lax.rem(nn, jnp.int32(t)) + sb * jnp.int32(c31) + hi * jnp.int32(c32)
    return lax.rem(acc, jnp.int32(t))


@functools.lru_cache(maxsize=1)
def _build():
    mesh = plsc.VectorSubcoreMesh(core_axis_name="c", subcore_axis_name="s")

    p1h, p1l = _P1 >> 16, _P1 & 0xFFFF
    p2h, p2l = _P2 >> 16, _P2 & 0xFFFF

    scratch = []
    for l in range(_NSMALL):
        scratch.append(pltpu.VMEM((_TS[l],), jnp.float32))
        scratch.append(pltpu.VMEM((_TS[l],), jnp.float32))
    scratch += [pltpu.VMEM((_C,), jnp.float32)] * 4
    scratch += [pltpu.VMEM((_C,), jnp.int32)] * 4
    scratch += [pltpu.VMEM((_C,), jnp.int32)] * 4
    scratch += [pltpu.VMEM((_C, 8), jnp.float32)] * 4
    scratch.append(pltpu.VMEM((_C, 2 * _NUM_LEVELS), jnp.float32))
    scratch.append(pltpu.SemaphoreType.DMA)

    @functools.partial(
        pl.kernel,
        out_type=jax.ShapeDtypeStruct((_BATCH, 2 * _NUM_LEVELS), jnp.float32),
        mesh=mesh,
        compiler_params=pltpu.CompilerParams(
            needs_layout_passes=False, use_tc_tiling_on_sc=False),
        scratch_types=scratch,
    )
    def hash_embed(*refs):
        xt_ref = refs[0]
        ta_hbm = refs[1:1 + _NSMALL]
        tb_hbm = refs[1 + _NSMALL:1 + 2 * _NSMALL]
        big_hbm = {_NSMALL + i: r for i, r in enumerate(
            refs[1 + 2 * _NSMALL:1 + 2 * _NSMALL + (_NUM_LEVELS - _NSMALL)])}
        out_ref = refs[1 + 2 * _NSMALL + (_NUM_LEVELS - _NSMALL)]
        sc = list(refs[2 + 2 * _NSMALL + (_NUM_LEVELS - _NSMALL):])
        st_a = [sc[2 * l] for l in range(_NSMALL)]
        st_b = [sc[2 * l + 1] for l in range(_NSMALL)]
        sc = sc[2 * _NSMALL:]
        xn, yn, wx, wy = sc[0:4]
        idx_refs = sc[4:8]
        sub_refs = sc[8:12]
        row_refs = sc[12:16]
        ob = sc[16]
        sem = sc[17]

        wid = (lax.axis_index("s").astype(jnp.int32) * jnp.int32(_NC)
               + lax.axis_index("c").astype(jnp.int32))
        iota = lax.iota(jnp.int32, _L)

        for l in range(_NSMALL):
            pltpu.sync_copy(ta_hbm[l], st_a[l])
            pltpu.sync_copy(tb_hbm[l], st_b[l])

        def level_consts(lvl):
            t = _TS[lvl]
            return dict(
                res_f=jnp.float32(_RES[lvl]),
                res_m1=jnp.int32(_RES[lvl] - 1),
                t=t,
                pow2=(t & (t - 1)) == 0,
                c31=(1 << 31) % t,
                c32=(1 << 32) % t,
                mask=jnp.int32(t - 1),
                p1c=_i32c(_P1),
                p2c=_i32c(_P2),
            )

        def corners(sl, lc):
            xnv = xn[sl]
            ynv = yn[sl]
            xs = xnv * lc["res_f"]
            ys = ynv * lc["res_f"]
            x0 = xs.astype(jnp.int32)
            y0 = ys.astype(jnp.int32)
            wxv = xs - x0.astype(jnp.float32)
            wyv = ys - y0.astype(jnp.float32)
            x1 = jnp.minimum(x0 + jnp.int32(1), lc["res_m1"])
            y1 = jnp.minimum(y0 + jnp.int32(1), lc["res_m1"])
            x0 = jnp.minimum(x0, lc["res_m1"])
            y0 = jnp.minimum(y0, lc["res_m1"])
            lox0 = x0 * lc["p1c"]
            lox1 = x1 * lc["p1c"]
            loy0 = y0 * lc["p2c"]
            loy1 = y1 * lc["p2c"]
            if lc["pow2"]:
                c00 = _combine_pow2(lox0, loy0, lc["mask"])
                c10 = _combine_pow2(lox1, loy0, lc["mask"])
                c01 = _combine_pow2(lox0, loy1, lc["mask"])
                c11 = _combine_pow2(lox1, loy1, lc["mask"])
            else:
                hix0 = _hash_hi(x0, p1h, p1l)
                hix1 = _hash_hi(x1, p1h, p1l)
                hiy0 = _hash_hi(y0, p2h, p2l)
                hiy1 = _hash_hi(y1, p2h, p2l)
                t, c31, c32 = lc["t"], lc["c31"], lc["c32"]
                c00 = _combine_general(lox0, hix0, loy0, hiy0, t, c31, c32)
                c10 = _combine_general(lox1, hix1, loy0, hiy0, t, c31, c32)
                c01 = _combine_general(lox0, hix0, loy1, hiy1, t, c31, c32)
                c11 = _combine_general(lox1, hix1, loy1, hiy1, t, c31, c32)
            return wxv, wyv, c00, c10, c01, c11

        def lerp2(e00, e10, e01, e11, wxv, wyv):
            top = e00 + (e10 - e00) * wxv
            bot = e01 + (e11 - e01) * wxv
            return top + (bot - top) * wyv

        def p0(i):
            s = pl.multiple_of(i, _L)
            sl = pl.ds(s, _L)
            xn[sl] = jnp.clip((xn[sl] + 1.0) * 0.5, 0.0, 1.0)
            yn[sl] = jnp.clip((yn[sl] + 1.0) * 0.5, 0.0, 1.0)

        def make_small(lvl):
            lc = level_consts(lvl)
            ca = jnp.full((_L,), 2 * lvl, jnp.int32)
            cb = jnp.full((_L,), 2 * lvl + 1, jnp.int32)
            ta, tb = st_a[lvl], st_b[lvl]

            def ps(i):
                s = pl.multiple_of(i, _L)
                sl = pl.ds(s, _L)
                pidx = s + iota
                wxv, wyv, c00, c10, c01, c11 = corners(sl, lc)
                e00a = plsc.load_gather(ta, [c00])
                e10a = plsc.load_gather(ta, [c10])
                e01a = plsc.load_gather(ta, [c01])
                e11a = plsc.load_gather(ta, [c11])
                e00b = plsc.load_gather(tb, [c00])
                e10b = plsc.load_gather(tb, [c10])
                e01b = plsc.load_gather(tb, [c01])
                e11b = plsc.load_gather(tb, [c11])
                plsc.store_scatter(ob, [pidx, ca],
                                   lerp2(e00a, e10a, e01a, e11a, wxv, wyv))
                plsc.store_scatter(ob, [pidx, cb],
                                   lerp2(e00b, e10b, e01b, e11b, wxv, wyv))

            return ps

        def make_p1(lvl):
            lc = level_consts(lvl)
            two = jnp.int32(2)
            three = jnp.int32(3)
            one = jnp.int32(1)

            def p1(i):
                s = pl.multiple_of(i, _L)
                sl = pl.ds(s, _L)
                wxv, wyv, c00, c10, c01, c11 = corners(sl, lc)
                wx[sl] = wxv
                wy[sl] = wyv
                for k, c in enumerate((c00, c10, c01, c11)):
                    idx_refs[k][sl] = lax.shift_right_logical(c, two)
                    sub_refs[k][sl] = lax.shift_left(c & three, one)

            return p1

        def make_p2(lvl):
            ca = jnp.full((_L,), 2 * lvl, jnp.int32)
            cb = jnp.full((_L,), 2 * lvl + 1, jnp.int32)
            one = jnp.int32(1)

            def p2(i):
                s = pl.multiple_of(i, _L)
                sl = pl.ds(s, _L)
                pidx = s + iota
                wxv = wx[sl]
                wyv = wy[sl]
                ea = []
                eb = []
                for k in range(4):
                    sv = sub_refs[k][sl]
                    ea.append(plsc.load_gather(row_refs[k], [iota + s, sv]))
                    eb.append(plsc.load_gather(row_refs[k], [iota + s, sv + one]))
                va = lerp2(ea[0], ea[1], ea[2], ea[3], wxv, wyv)
                vb = lerp2(eb[0], eb[1], eb[2], eb[3], wxv, wyv)
                plsc.store_scatter(ob, [pidx, ca], va)
                plsc.store_scatter(ob, [pidx, cb], vb)

            return p2

        smalls = [make_small(l) for l in range(_NSMALL)]
        p1s = {l: make_p1(l) for l in range(_NSMALL, _NUM_LEVELS)}
        p2s = {l: make_p2(l) for l in range(_NSMALL, _NUM_LEVELS)}

        def chunk(g, carry):
            base = pl.multiple_of(wid * jnp.int32(_PW) + g * jnp.int32(_C), _C)
            pltpu.sync_copy(xt_ref.at[jnp.int32(0), pl.ds(base, _C)], xn)
            pltpu.sync_copy(xt_ref.at[jnp.int32(1), pl.ds(base, _C)], yn)
            plsc.parallel_loop(jnp.int32(0), jnp.int32(_C), jnp.int32(_L), unroll=2)(p0)
            for l in range(_NSMALL):
                plsc.parallel_loop(jnp.int32(0), jnp.int32(_C), jnp.int32(_L), unroll=2)(smalls[l])
            for l in range(_NSMALL, _NUM_LEVELS):
                plsc.parallel_loop(jnp.int32(0), jnp.int32(_C), jnp.int32(_L), unroll=2)(p1s[l])
                cps = [pltpu.async_copy(big_hbm[l].at[idx_refs[k]],
                                        row_refs[k], sem)
                       for k in range(4)]
                for cp in cps:
                    cp.wait()
                plsc.parallel_loop(jnp.int32(0), jnp.int32(_C), jnp.int32(_L), unroll=2)(p2s[l])
            pltpu.sync_copy(ob, out_ref.at[pl.ds(base, _C)])
            return carry

        lax.fori_loop(jnp.int32(0), jnp.int32(_NCHUNK), chunk, 0)

    return hash_embed


def kernel(x, tables):
    xt = x.T
    tas = tuple(tables[l][:, 0] for l in range(_NSMALL))
    tbs = tuple(tables[l][:, 1] for l in range(_NSMALL))
    bigs = []
    for l in range(_NSMALL, _NUM_LEVELS):
        flat = tables[l].reshape(-1)
        pad = (-flat.shape[0]) % 8
        if pad:
            flat = jnp.concatenate([flat, jnp.zeros((pad,), flat.dtype)])
        bigs.append(flat.reshape(-1, 8))
    return _build()(xt, *tas, *tbs, *bigs)


# speedup vs baseline: 1.8011x; 1.6182x over previous
import functools

import numpy as np
import jax
import jax.numpy as jnp
from jax import lax
from jax.experimental import pallas as pl
from jax.experimental.pallas import tpu as pltpu
from jax.experimental.pallas import tpu_sc as plsc

_NUM_LEVELS = 16
_BASE_RES = 16
_MAX_RES = 2048
_HASHMAP_SIZE = 2 ** 19
_GROWTH = np.exp((np.log(_MAX_RES) - np.log(_BASE_RES)) / (_NUM_LEVELS - 1))
_RES = [int(np.floor(_BASE_RES * _GROWTH ** i)) for i in range(_NUM_LEVELS)]
_TS = [min(r * r, _HASHMAP_SIZE) for r in _RES]
_P1 = 2654435761
_P2 = 805459861
_BATCH = 1048576

_NC, _NS, _L = 2, 16, 16
_NW = _NC * _NS
_PW = _BATCH // _NW
_C = 1024
_NCHUNK = _PW // _C
_NSMALL = 7


def _i32c(v):
    return jnp.int32(np.int32(np.uint32(v & 0xFFFFFFFF)))


def _hash_hi(v, ph, plo):
    c16 = jnp.int32(16)
    a = v * ph
    b = v * plo
    return lax.shift_right_logical(a + lax.shift_right_logical(b, c16), c16)


def _combine_pow2(lox, loy, mask):
    return (lox ^ loy) & mask


def _combine_general(lox, hix, loy, hiy, t, c31, c32):
    lo = lox ^ loy
    hi = hix ^ hiy
    nn = lo & jnp.int32(0x7FFFFFFF)
    sb = lax.shift_right_logical(lo, jnp.int32(31))
    acc = lax.rem(nn, jnp.int32(t)) + sb * jnp.int32(c31) + hi * jnp.int32(c32)
    return lax.rem(acc, jnp.int32(t))


@functools.lru_cache(maxsize=1)
def _build():
    mesh = plsc.VectorSubcoreMesh(core_axis_name="c", subcore_axis_name="s")

    p1h, p1l = _P1 >> 16, _P1 & 0xFFFF
    p2h, p2l = _P2 >> 16, _P2 & 0xFFFF

    scratch = []
    for l in range(_NSMALL):
        scratch.append(pltpu.VMEM((_TS[l],), jnp.float32))
        scratch.append(pltpu.VMEM((_TS[l],), jnp.float32))
    scratch += [pltpu.VMEM((_C,), jnp.float32)] * 2
    for _ in range(2):
        scratch += [pltpu.VMEM((_C,), jnp.float32)] * 2
        scratch += [pltpu.VMEM((_C,), jnp.int32)] * 4
        scratch += [pltpu.VMEM((_C,), jnp.float32)] * 8
        scratch.append(pltpu.SemaphoreType.DMA)
    scratch.append(pltpu.VMEM((_C, 2 * _NUM_LEVELS), jnp.float32))

    @functools.partial(
        pl.kernel,
        out_type=jax.ShapeDtypeStruct((_BATCH, 2 * _NUM_LEVELS), jnp.float32),
        mesh=mesh,
        compiler_params=pltpu.CompilerParams(
            needs_layout_passes=False, use_tc_tiling_on_sc=False),
        scratch_types=scratch,
    )
    def hash_embed(*refs):
        xt_ref = refs[0]
        ta_hbm = refs[1:1 + _NUM_LEVELS]
        tb_hbm = refs[1 + _NUM_LEVELS:1 + 2 * _NUM_LEVELS]
        out_ref = refs[1 + 2 * _NUM_LEVELS]
        sc = list(refs[2 + 2 * _NUM_LEVELS:])
        st_a = [sc[2 * l] for l in range(_NSMALL)]
        st_b = [sc[2 * l + 1] for l in range(_NSMALL)]
        sc = sc[2 * _NSMALL:]
        xn, yn = sc[0:2]
        slots = []
        p = 2
        for _ in range(2):
            slots.append(dict(
                wx=sc[p], wy=sc[p + 1],
                idx=sc[p + 2:p + 6],
                rows_a=sc[p + 6:p + 10],
                rows_b=sc[p + 10:p + 14],
                sem=sc[p + 14],
            ))
            p += 15
        ob = sc[p]

        wid = (lax.axis_index("s").astype(jnp.int32) * jnp.int32(_NC)
               + lax.axis_index("c").astype(jnp.int32))
        iota = lax.iota(jnp.int32, _L)

        for l in range(_NSMALL):
            pltpu.sync_copy(ta_hbm[l], st_a[l])
            pltpu.sync_copy(tb_hbm[l], st_b[l])

        def level_consts(lvl):
            t = _TS[lvl]
            return dict(
                res_f=jnp.float32(_RES[lvl]),
                res_m1=jnp.int32(_RES[lvl] - 1),
                t=t,
                pow2=(t & (t - 1)) == 0,
                c31=(1 << 31) % t,
                c32=(1 << 32) % t,
                mask=jnp.int32(t - 1),
                p1c=_i32c(_P1),
                p2c=_i32c(_P2),
            )

        def corners(sl, lc):
            xnv = xn[sl]
            ynv = yn[sl]
            xs = xnv * lc["res_f"]
            ys = ynv * lc["res_f"]
            x0 = xs.astype(jnp.int32)
            y0 = ys.astype(jnp.int32)
            wxv = xs - x0.astype(jnp.float32)
            wyv = ys - y0.astype(jnp.float32)
            x1 = jnp.minimum(x0 + jnp.int32(1), lc["res_m1"])
            y1 = jnp.minimum(y0 + jnp.int32(1), lc["res_m1"])
            x0 = jnp.minimum(x0, lc["res_m1"])
            y0 = jnp.minimum(y0, lc["res_m1"])
            lox0 = x0 * lc["p1c"]
            lox1 = x1 * lc["p1c"]
            loy0 = y0 * lc["p2c"]
            loy1 = y1 * lc["p2c"]
            if lc["pow2"]:
                c00 = _combine_pow2(lox0, loy0, lc["mask"])
                c10 = _combine_pow2(lox1, loy0, lc["mask"])
                c01 = _combine_pow2(lox0, loy1, lc["mask"])
                c11 = _combine_pow2(lox1, loy1, lc["mask"])
            else:
                hix0 = _hash_hi(x0, p1h, p1l)
                hix1 = _hash_hi(x1, p1h, p1l)
                hiy0 = _hash_hi(y0, p2h, p2l)
                hiy1 = _hash_hi(y1, p2h, p2l)
                t, c31, c32 = lc["t"], lc["c31"], lc["c32"]
                c00 = _combine_general(lox0, hix0, loy0, hiy0, t, c31, c32)
                c10 = _combine_general(lox1, hix1, loy0, hiy0, t, c31, c32)
                c01 = _combine_general(lox0, hix0, loy1, hiy1, t, c31, c32)
                c11 = _combine_general(lox1, hix1, loy1, hiy1, t, c31, c32)
            return wxv, wyv, c00, c10, c01, c11

        def lerp2(e00, e10, e01, e11, wxv, wyv):
            top = e00 + (e10 - e00) * wxv
            bot = e01 + (e11 - e01) * wxv
            return top + (bot - top) * wyv

        def ploop(body):
            plsc.parallel_loop(jnp.int32(0), jnp.int32(_C), jnp.int32(_L),
                               unroll=2)(body)

        def p0(i):
            s = pl.multiple_of(i, _L)
            sl = pl.ds(s, _L)
            xn[sl] = jnp.clip((xn[sl] + 1.0) * 0.5, 0.0, 1.0)
            yn[sl] = jnp.clip((yn[sl] + 1.0) * 0.5, 0.0, 1.0)

        def make_small(lvl):
            lc = level_consts(lvl)
            ca = jnp.full((_L,), 2 * lvl, jnp.int32)
            cb = jnp.full((_L,), 2 * lvl + 1, jnp.int32)
            ta, tb = st_a[lvl], st_b[lvl]

            def ps(i):
                s = pl.multiple_of(i, _L)
                sl = pl.ds(s, _L)
                pidx = s + iota
                wxv, wyv, c00, c10, c01, c11 = corners(sl, lc)
                e00a = plsc.load_gather(ta, [c00])
                e10a = plsc.load_gather(ta, [c10])
                e01a = plsc.load_gather(ta, [c01])
                e11a = plsc.load_gather(ta, [c11])
                e00b = plsc.load_gather(tb, [c00])
                e10b = plsc.load_gather(tb, [c10])
                e01b = plsc.load_gather(tb, [c01])
                e11b = plsc.load_gather(tb, [c11])
                plsc.store_scatter(ob, [pidx, ca],
                                   lerp2(e00a, e10a, e01a, e11a, wxv, wyv))
                plsc.store_scatter(ob, [pidx, cb],
                                   lerp2(e00b, e10b, e01b, e11b, wxv, wyv))

            return ps

        def make_p1(lvl, slot):
            lc = level_consts(lvl)
            wxr, wyr = slot["wx"], slot["wy"]
            idx = slot["idx"]

            def p1(i):
                s = pl.multiple_of(i, _L)
                sl = pl.ds(s, _L)
                wxv, wyv, c00, c10, c01, c11 = corners(sl, lc)
                wxr[sl] = wxv
                wyr[sl] = wyv
                idx[0][sl] = c00
                idx[1][sl] = c10
                idx[2][sl] = c01
                idx[3][sl] = c11

            return p1

        def fire(lvl, slot):
            cps = []
            for k in range(4):
                cps.append(pltpu.async_copy(
                    ta_hbm[lvl].at[slot["idx"][k]], slot["rows_a"][k],
                    slot["sem"]))
                cps.append(pltpu.async_copy(
                    tb_hbm[lvl].at[slot["idx"][k]], slot["rows_b"][k],
                    slot["sem"]))
            return cps

        def make_p2(lvl, slot):
            ca = jnp.full((_L,), 2 * lvl, jnp.int32)
            cb = jnp.full((_L,), 2 * lvl + 1, jnp.int32)
            ra, rb = slot["rows_a"], slot["rows_b"]
            wxr, wyr = slot["wx"], slot["wy"]

            def p2(i):
                s = pl.multiple_of(i, _L)
                sl = pl.ds(s, _L)
                pidx = s + iota
                wxv = wxr[sl]
                wyv = wyr[sl]
                va = lerp2(ra[0][sl], ra[1][sl], ra[2][sl], ra[3][sl],
                           wxv, wyv)
                vb = lerp2(rb[0][sl], rb[1][sl], rb[2][sl], rb[3][sl],
                           wxv, wyv)
                plsc.store_scatter(ob, [pidx, ca], va)
                plsc.store_scatter(ob, [pidx, cb], vb)

            return p2

        smalls = [make_small(l) for l in range(_NSMALL)]
        p1s = {l: make_p1(l, slots[(l - _NSMALL) % 2])
               for l in range(_NSMALL, _NUM_LEVELS)}
        p2s = {l: make_p2(l, slots[(l - _NSMALL) % 2])
               for l in range(_NSMALL, _NUM_LEVELS)}

        def chunk(g, carry):
            base = pl.multiple_of(wid * jnp.int32(_PW) + g * jnp.int32(_C), _C)
            pltpu.sync_copy(xt_ref.at[jnp.int32(0), pl.ds(base, _C)], xn)
            pltpu.sync_copy(xt_ref.at[jnp.int32(1), pl.ds(base, _C)], yn)
            ploop(p0)
            ploop(p1s[_NSMALL])
            inflight = {_NSMALL: fire(_NSMALL, slots[0])}
            ploop(p1s[_NSMALL + 1])
            inflight[_NSMALL + 1] = fire(_NSMALL + 1, slots[1])
            for l in range(_NSMALL):
                ploop(smalls[l])
            for l in range(_NSMALL, _NUM_LEVELS):
                for cp in inflight.pop(l):
                    cp.wait()
                ploop(p2s[l])
                nxt = l + 2
                if nxt < _NUM_LEVELS:
                    ploop(p1s[nxt])
                    inflight[nxt] = fire(nxt, slots[(nxt - _NSMALL) % 2])
            pltpu.sync_copy(ob, out_ref.at[pl.ds(base, _C)])
            return carry

        lax.fori_loop(jnp.int32(0), jnp.int32(_NCHUNK), chunk, 0)

    return hash_embed


def kernel(x, tables):
    xt = x.T
    tas = tuple(t[:, 0] for t in tables)
    tbs = tuple(t[:, 1] for t in tables)
    return _build()(xt, *tas, *tbs)


# speedup vs baseline: 1.8899x; 1.0493x over previous
import functools

import numpy as np
import jax
import jax.numpy as jnp
from jax import lax
from jax.experimental import pallas as pl
from jax.experimental.pallas import tpu as pltpu
from jax.experimental.pallas import tpu_sc as plsc

_NUM_LEVELS = 16
_BASE_RES = 16
_MAX_RES = 2048
_HASHMAP_SIZE = 2 ** 19
_GROWTH = np.exp((np.log(_MAX_RES) - np.log(_BASE_RES)) / (_NUM_LEVELS - 1))
_RES = [int(np.floor(_BASE_RES * _GROWTH ** i)) for i in range(_NUM_LEVELS)]
_TS = [min(r * r, _HASHMAP_SIZE) for r in _RES]
_P1 = 2654435761
_P2 = 805459861
_BATCH = 1048576

_NC, _NS, _L = 2, 16, 16
_NW = _NC * _NS
_PW = _BATCH // _NW
_C = 1024
_NCHUNK = _PW // _C
_NSMALL = 7


def _i32c(v):
    return jnp.int32(np.int32(np.uint32(v & 0xFFFFFFFF)))


def _hash_hi(v, ph, plo):
    c16 = jnp.int32(16)
    a = v * ph
    b = v * plo
    return lax.shift_right_logical(a + lax.shift_right_logical(b, c16), c16)


def _combine_pow2(lox, loy, mask):
    return (lox ^ loy) & mask


def _combine_general(lox, hix, loy, hiy, t, c31, c32):
    lo = lox ^ loy
    hi = hix ^ hiy
    nn = lo & jnp.int32(0x7FFFFFFF)
    sb = lax.shift_right_logical(lo, jnp.int32(31))
    m = t * 256
    q1 = (nn.astype(jnp.float32) * jnp.float32(1.0 / m)).astype(jnp.int32)
    r1 = nn - q1 * jnp.int32(m)
    r1 = jnp.where(r1 < jnp.int32(0), r1 + jnp.int32(m), r1)
    acc = r1 + sb * jnp.int32(c31) + hi * jnp.int32(c32)
    q2 = (acc.astype(jnp.float32) * jnp.float32(1.0 / t)).astype(jnp.int32)
    r2 = acc - q2 * jnp.int32(t)
    r2 = jnp.where(r2 < jnp.int32(0), r2 + jnp.int32(t), r2)
    r2 = jnp.where(r2 >= jnp.int32(t), r2 - jnp.int32(t), r2)
    return r2


@functools.lru_cache(maxsize=1)
def _build():
    mesh = plsc.VectorSubcoreMesh(core_axis_name="c", subcore_axis_name="s")

    p1h, p1l = _P1 >> 16, _P1 & 0xFFFF
    p2h, p2l = _P2 >> 16, _P2 & 0xFFFF

    scratch = []
    for l in range(_NSMALL):
        scratch.append(pltpu.VMEM((_TS[l],), jnp.float32))
        scratch.append(pltpu.VMEM((_TS[l],), jnp.float32))
    scratch += [pltpu.VMEM((_C,), jnp.float32)] * 2
    for _ in range(2):
        scratch += [pltpu.VMEM((_C,), jnp.float32)] * 2
        scratch += [pltpu.VMEM((_C,), jnp.int32)] * 4
        scratch += [pltpu.VMEM((_C,), jnp.float32)] * 8
        scratch.append(pltpu.SemaphoreType.DMA)
    scratch.append(pltpu.VMEM((_C, 2 * _NUM_LEVELS), jnp.float32))

    @functools.partial(
        pl.kernel,
        out_type=jax.ShapeDtypeStruct((_BATCH, 2 * _NUM_LEVELS), jnp.float32),
        mesh=mesh,
        compiler_params=pltpu.CompilerParams(
            needs_layout_passes=False, use_tc_tiling_on_sc=False),
        scratch_types=scratch,
    )
    def hash_embed(*refs):
        xt_ref = refs[0]
        ta_hbm = refs[1:1 + _NUM_LEVELS]
        tb_hbm = refs[1 + _NUM_LEVELS:1 + 2 * _NUM_LEVELS]
        out_ref = refs[1 + 2 * _NUM_LEVELS]
        sc = list(refs[2 + 2 * _NUM_LEVELS:])
        st_a = [sc[2 * l] for l in range(_NSMALL)]
        st_b = [sc[2 * l + 1] for l in range(_NSMALL)]
        sc = sc[2 * _NSMALL:]
        xn, yn = sc[0:2]
        slots = []
        p = 2
        for _ in range(2):
            slots.append(dict(
                wx=sc[p], wy=sc[p + 1],
                idx=sc[p + 2:p + 6],
                rows_a=sc[p + 6:p + 10],
                rows_b=sc[p + 10:p + 14],
                sem=sc[p + 14],
            ))
            p += 15
        ob = sc[p]

        wid = (lax.axis_index("s").astype(jnp.int32) * jnp.int32(_NC)
               + lax.axis_index("c").astype(jnp.int32))
        iota = lax.iota(jnp.int32, _L)

        for l in range(_NSMALL):
            pltpu.sync_copy(ta_hbm[l], st_a[l])
            pltpu.sync_copy(tb_hbm[l], st_b[l])

        def level_consts(lvl):
            t = _TS[lvl]
            return dict(
                res_f=jnp.float32(_RES[lvl]),
                res_m1=jnp.int32(_RES[lvl] - 1),
                t=t,
                pow2=(t & (t - 1)) == 0,
                c31=(1 << 31) % t,
                c32=(1 << 32) % t,
                mask=jnp.int32(t - 1),
                p1c=_i32c(_P1),
                p2c=_i32c(_P2),
            )

        def corners(sl, lc):
            xnv = xn[sl]
            ynv = yn[sl]
            xs = xnv * lc["res_f"]
            ys = ynv * lc["res_f"]
            x0 = xs.astype(jnp.int32)
            y0 = ys.astype(jnp.int32)
            wxv = xs - x0.astype(jnp.float32)
            wyv = ys - y0.astype(jnp.float32)
            x1 = jnp.minimum(x0 + jnp.int32(1), lc["res_m1"])
            y1 = jnp.minimum(y0 + jnp.int32(1), lc["res_m1"])
            x0 = jnp.minimum(x0, lc["res_m1"])
            y0 = jnp.minimum(y0, lc["res_m1"])
            lox0 = x0 * lc["p1c"]
            lox1 = x1 * lc["p1c"]
            loy0 = y0 * lc["p2c"]
            loy1 = y1 * lc["p2c"]
            if lc["pow2"]:
                c00 = _combine_pow2(lox0, loy0, lc["mask"])
                c10 = _combine_pow2(lox1, loy0, lc["mask"])
                c01 = _combine_pow2(lox0, loy1, lc["mask"])
                c11 = _combine_pow2(lox1, loy1, lc["mask"])
            else:
                hix0 = _hash_hi(x0, p1h, p1l)
                hix1 = _hash_hi(x1, p1h, p1l)
                hiy0 = _hash_hi(y0, p2h, p2l)
                hiy1 = _hash_hi(y1, p2h, p2l)
                t, c31, c32 = lc["t"], lc["c31"], lc["c32"]
                c00 = _combine_general(lox0, hix0, loy0, hiy0, t, c31, c32)
                c10 = _combine_general(lox1, hix1, loy0, hiy0, t, c31, c32)
                c01 = _combine_general(lox0, hix0, loy1, hiy1, t, c31, c32)
                c11 = _combine_general(lox1, hix1, loy1, hiy1, t, c31, c32)
            return wxv, wyv, c00, c10, c01, c11

        def lerp2(e00, e10, e01, e11, wxv, wyv):
            top = e00 + (e10 - e00) * wxv
            bot = e01 + (e11 - e01) * wxv
            return top + (bot - top) * wyv

        def ploop(body):
            plsc.parallel_loop(jnp.int32(0), jnp.int32(_C), jnp.int32(_L),
                               unroll=2)(body)

        def p0(i):
            s = pl.multiple_of(i, _L)
            sl = pl.ds(s, _L)
            xn[sl] = jnp.clip((xn[sl] + 1.0) * 0.5, 0.0, 1.0)
            yn[sl] = jnp.clip((yn[sl] + 1.0) * 0.5, 0.0, 1.0)

        def make_small(lvl):
            lc = level_consts(lvl)
            ca = jnp.full((_L,), 2 * lvl, jnp.int32)
            cb = jnp.full((_L,), 2 * lvl + 1, jnp.int32)
            ta, tb = st_a[lvl], st_b[lvl]

            def ps(i):
                s = pl.multiple_of(i, _L)
                sl = pl.ds(s, _L)
                pidx = s + iota
                wxv, wyv, c00, c10, c01, c11 = corners(sl, lc)
                e00a = plsc.load_gather(ta, [c00])
                e10a = plsc.load_gather(ta, [c10])
                e01a = plsc.load_gather(ta, [c01])
                e11a = plsc.load_gather(ta, [c11])
                e00b = plsc.load_gather(tb, [c00])
                e10b = plsc.load_gather(tb, [c10])
                e01b = plsc.load_gather(tb, [c01])
                e11b = plsc.load_gather(tb, [c11])
                plsc.store_scatter(ob, [pidx, ca],
                                   lerp2(e00a, e10a, e01a, e11a, wxv, wyv))
                plsc.store_scatter(ob, [pidx, cb],
                                   lerp2(e00b, e10b, e01b, e11b, wxv, wyv))

            return ps

        def make_p1(lvl, slot):
            lc = level_consts(lvl)
            wxr, wyr = slot["wx"], slot["wy"]
            idx = slot["idx"]

            def p1(i):
                s = pl.multiple_of(i, _L)
                sl = pl.ds(s, _L)
                wxv, wyv, c00, c10, c01, c11 = corners(sl, lc)
                wxr[sl] = wxv
                wyr[sl] = wyv
                idx[0][sl] = c00
                idx[1][sl] = c10
                idx[2][sl] = c01
                idx[3][sl] = c11

            return p1

        def fire(lvl, slot):
            cps = []
            for k in range(4):
                cps.append(pltpu.async_copy(
                    ta_hbm[lvl].at[slot["idx"][k]], slot["rows_a"][k],
                    slot["sem"]))
                cps.append(pltpu.async_copy(
                    tb_hbm[lvl].at[slot["idx"][k]], slot["rows_b"][k],
                    slot["sem"]))
            return cps

        def make_p2(lvl, slot):
            ca = jnp.full((_L,), 2 * lvl, jnp.int32)
            cb = jnp.full((_L,), 2 * lvl + 1, jnp.int32)
            ra, rb = slot["rows_a"], slot["rows_b"]
            wxr, wyr = slot["wx"], slot["wy"]

            def p2(i):
                s = pl.multiple_of(i, _L)
                sl = pl.ds(s, _L)
                pidx = s + iota
                wxv = wxr[sl]
                wyv = wyr[sl]
                va = lerp2(ra[0][sl], ra[1][sl], ra[2][sl], ra[3][sl],
                           wxv, wyv)
                vb = lerp2(rb[0][sl], rb[1][sl], rb[2][sl], rb[3][sl],
                           wxv, wyv)
                plsc.store_scatter(ob, [pidx, ca], va)
                plsc.store_scatter(ob, [pidx, cb], vb)

            return p2

        smalls = [make_small(l) for l in range(_NSMALL)]
        p1s = {l: make_p1(l, slots[(l - _NSMALL) % 2])
               for l in range(_NSMALL, _NUM_LEVELS)}
        p2s = {l: make_p2(l, slots[(l - _NSMALL) % 2])
               for l in range(_NSMALL, _NUM_LEVELS)}

        def chunk(g, carry):
            base = pl.multiple_of(wid * jnp.int32(_PW) + g * jnp.int32(_C), _C)
            pltpu.sync_copy(xt_ref.at[jnp.int32(0), pl.ds(base, _C)], xn)
            pltpu.sync_copy(xt_ref.at[jnp.int32(1), pl.ds(base, _C)], yn)
            ploop(p0)
            ploop(p1s[_NSMALL])
            inflight = {_NSMALL: fire(_NSMALL, slots[0])}
            ploop(p1s[_NSMALL + 1])
            inflight[_NSMALL + 1] = fire(_NSMALL + 1, slots[1])
            for l in range(_NSMALL):
                ploop(smalls[l])
            for l in range(_NSMALL, _NUM_LEVELS):
                for cp in inflight.pop(l):
                    cp.wait()
                ploop(p2s[l])
                nxt = l + 2
                if nxt < _NUM_LEVELS:
                    ploop(p1s[nxt])
                    inflight[nxt] = fire(nxt, slots[(nxt - _NSMALL) % 2])
            pltpu.sync_copy(ob, out_ref.at[pl.ds(base, _C)])
            return carry

        lax.fori_loop(jnp.int32(0), jnp.int32(_NCHUNK), chunk, 0)

    return hash_embed


def kernel(x, tables):
    xt = x.T
    tas = tuple(t[:, 0] for t in tables)
    tbs = tuple(t[:, 1] for t in tables)
    return _build()(xt, *tas, *tbs)
